# baseline (speedup 1.0000x reference)
import functools

import jax
import jax.numpy as jnp
from jax import lax
from jax.experimental import pallas as pl
from jax.experimental.pallas import tpu as pltpu
from jax.experimental.pallas import tpu_sc as plsc

B, C, K = 4096, 20, 5
D, VDIM = 64, 16
G, SL = 125000, 8

NC, NS = 2, 16
NW = NC * NS
CH = 16
L = 16


NB = 4


def _gather_rows_body(n_rows, tab_hbm, gi_hbm, out_hbm,
                      giv, buf, gsems, osems):
    per_w = n_rows // NW
    n_ch = per_w // CH
    wid = lax.axis_index("s") * NC + lax.axis_index("c")
    base = wid * per_w
    pltpu.sync_copy(gi_hbm.at[pl.ds(base, per_w)], giv)

    def fire_gather(j, b):
        gvec = giv[pl.ds(pl.multiple_of(j * CH, L), L)]
        for i in range(CH):
            pltpu.async_copy(tab_hbm.at[gvec[i]], buf.at[b, i], gsems.at[b])

    def wait_gather(b):
        pltpu.make_async_copy(tab_hbm.at[pl.ds(0, CH)], buf.at[b],
                              gsems.at[b]).wait()

    def fire_wb(j, b):
        dst = out_hbm.at[pl.ds(pl.multiple_of(base + j * CH, CH), CH)]
        return pltpu.async_copy(buf.at[b], dst, osems.at[b])

    def wait_wb(b):
        pltpu.make_async_copy(out_hbm.at[pl.ds(0, CH)], buf.at[b],
                              osems.at[b]).wait()

    for j in range(NB - 1):
        fire_gather(j, j)

    def loop_body(j2, carry):
        for b in range(NB):
            j = j2 * NB + b
            bn = (b + NB - 1) % NB
            wait_gather(b)
            fire_wb(j, b)

            @pl.when(j + NB - 1 < n_ch)
            def _():
                @pl.when(j > 0)
                def _():
                    wait_wb(bn)
                fire_gather(j + NB - 1, bn)
        return carry

    lax.fori_loop(0, n_ch // NB, loop_body, 0)
    for b in range(NB):
        wait_wb(b)


def _sc_gather_table(n_rows, tab, gi):
    body = functools.partial(_gather_rows_body, n_rows)
    return pl.kernel(
        body,
        out_type=jax.ShapeDtypeStruct((n_rows, D), jnp.float32),
        mesh=plsc.VectorSubcoreMesh(core_axis_name="c", subcore_axis_name="s"),
        compiler_params=pltpu.CompilerParams(use_tc_tiling_on_sc=True,
                                             needs_layout_passes=False),
        scratch_types=[
            pltpu.VMEM((n_rows // NW,), jnp.int32),
            pltpu.VMEM((NB, CH, D), jnp.float32),
            pltpu.SemaphoreType.DMA((NB,)),
            pltpu.SemaphoreType.DMA((NB,)),
        ],
    )(tab, gi)


BB = 256
GRID = B // BB


def _log_sigmoid(x):
    return jnp.minimum(x, 0.0) - jnp.log(1.0 + jnp.exp(-jnp.abs(x)))


def _tc_body(vp_ref, gu_ref, gp_ref, gn_ref, out_ref):
    @pl.when(pl.program_id(0) == 0)
    def _init():
        out_ref[0, 0] = 0.0

    vp = vp_ref[...]
    vp2 = vp * vp
    dn = (((1,), (1,)), ((), ()))
    S = jnp.zeros((BB, D), jnp.float32)
    acc = jnp.zeros((BB, 1), jnp.float32)
    for c in range(C):
        ec = gu_ref[c]
        t = lax.dot_general(ec, vp, dn, preferred_element_type=jnp.float32)
        t2 = lax.dot_general(ec * ec, vp2, dn, preferred_element_type=jnp.float32)
        acc = acc + jnp.sum(t * t - t2, axis=1, keepdims=True)
        S = S + ec
    fm = 0.5 * acc
    pu = S + C * fm
    s2 = jnp.sum(pu * gp_ref[...], axis=1, keepdims=True)
    nsum = jnp.zeros((BB, D), jnp.float32)
    for k in range(K):
        nsum = nsum + gn_ref[k]
    ns2 = jnp.sum(nsum * pu, axis=1, keepdims=True)
    part = jnp.sum(_log_sigmoid(s2)) + jnp.sum(_log_sigmoid(-ns2))
    out_ref[0, 0] += part


def _tc_score(Vp, gu3, gp, gn3):
    return pl.pallas_call(
        _tc_body,
        grid=(GRID,),
        in_specs=[
            pl.BlockSpec((VDIM, D), lambda i: (0, 0)),
            pl.BlockSpec((C, BB, D), lambda i: (0, i, 0)),
            pl.BlockSpec((BB, D), lambda i: (i, 0)),
            pl.BlockSpec((K, BB, D), lambda i: (0, i, 0)),
        ],
        out_specs=pl.BlockSpec((1, 1), lambda i: (0, 0),
                               memory_space=pltpu.SMEM),
        out_shape=jax.ShapeDtypeStruct((1, 1), jnp.float32),
    )(Vp, gu3, gp, gn3)


def kernel(pos_u, pos_w, neg_w, U, W, Vp):
    iu_t = pos_u.astype(jnp.int32).T.reshape(-1)
    in_t = neg_w.astype(jnp.int32).T.reshape(-1)
    iw_t = jnp.concatenate([pos_w.astype(jnp.int32), in_t])
    gu = _sc_gather_table(C * B, U, iu_t)
    gw = _sc_gather_table(B + K * B, W, iw_t)
    out = _tc_score(Vp, gu.reshape(C, B, D), gw[:B],
                    gw[B:].reshape(K, B, D))
    return -out[0, 0]


# speedup vs baseline: 1.0145x; 1.0145x over previous
import functools

import jax
import jax.numpy as jnp
from jax import lax
from jax.experimental import pallas as pl
from jax.experimental.pallas import tpu as pltpu
from jax.experimental.pallas import tpu_sc as plsc

B, C, K = 4096, 20, 5
D, VDIM = 64, 16

NC, NS = 2, 16
NW = NC * NS
CH = 16
L = 16

NB = 8
PD = 4


def _gather_rows_body(n_rows, tab_hbm, gi_hbm, out_hbm,
                      giv, buf, gsems, osems):
    per_w = n_rows // NW
    n_ch = per_w // CH
    wid = lax.axis_index("s") * NC + lax.axis_index("c")
    base = wid * per_w
    pltpu.sync_copy(gi_hbm.at[pl.ds(base, per_w)], giv)

    def fire_gather(j, b):
        gvec = giv[pl.ds(pl.multiple_of(j * CH, L), L)]
        for i in range(CH):
            pltpu.async_copy(tab_hbm.at[gvec[i]], buf.at[b, i], gsems.at[b])

    def wait_gather(b):
        pltpu.make_async_copy(tab_hbm.at[pl.ds(0, CH)], buf.at[b],
                              gsems.at[b]).wait()

    def fire_wb(j, b):
        dst = out_hbm.at[pl.ds(pl.multiple_of(base + j * CH, CH), CH)]
        return pltpu.async_copy(buf.at[b], dst, osems.at[b])

    def wait_wb(b):
        pltpu.make_async_copy(out_hbm.at[pl.ds(0, CH)], buf.at[b],
                              osems.at[b]).wait()

    for j in range(PD):
        fire_gather(j, j)

    def loop_body(j2, carry):
        for b in range(NB):
            j = j2 * NB + b
            bn = (b + PD) % NB
            wait_gather(b)
            fire_wb(j, b)

            @pl.when(j + PD < n_ch)
            def _():
                @pl.when(j + PD >= NB)
                def _():
                    wait_wb(bn)
                fire_gather(j + PD, bn)
        return carry

    lax.fori_loop(0, n_ch // NB, loop_body, 0)
    for b in range(NB):
        wait_wb(b)


def _sc_gather_table(n_rows, tab, gi):
    body = functools.partial(_gather_rows_body, n_rows)
    return pl.kernel(
        body,
        out_type=jax.ShapeDtypeStruct((n_rows, D), jnp.float32),
        mesh=plsc.VectorSubcoreMesh(core_axis_name="c", subcore_axis_name="s"),
        compiler_params=pltpu.CompilerParams(use_tc_tiling_on_sc=True,
                                             needs_layout_passes=False),
        scratch_types=[
            pltpu.VMEM((n_rows // NW,), jnp.int32),
            pltpu.VMEM((NB, CH, D), jnp.float32),
            pltpu.SemaphoreType.DMA((NB,)),
            pltpu.SemaphoreType.DMA((NB,)),
        ],
    )(tab, gi)


BB = 256
GRID = B // BB


def _log_sigmoid(x):
    return jnp.minimum(x, 0.0) - jnp.log(1.0 + jnp.exp(-jnp.abs(x)))


def _tc_body(vp_ref, gu_ref, gp_ref, gn_ref, out_ref):
    @pl.when(pl.program_id(0) == 0)
    def _init():
        out_ref[0, 0] = 0.0

    vp = vp_ref[...]
    vp2 = vp * vp
    dn = (((1,), (1,)), ((), ()))
    S = jnp.zeros((BB, D), jnp.float32)
    acc = jnp.zeros((BB, 1), jnp.float32)
    for c in range(C):
        ec = gu_ref[c]
        t = lax.dot_general(ec, vp, dn, preferred_element_type=jnp.float32)
        t2 = lax.dot_general(ec * ec, vp2, dn, preferred_element_type=jnp.float32)
        acc = acc + jnp.sum(t * t - t2, axis=1, keepdims=True)
        S = S + ec
    fm = 0.5 * acc
    pu = S + C * fm
    s2 = jnp.sum(pu * gp_ref[...], axis=1, keepdims=True)
    nsum = jnp.zeros((BB, D), jnp.float32)
    for k in range(K):
        nsum = nsum + gn_ref[k]
    ns2 = jnp.sum(nsum * pu, axis=1, keepdims=True)
    part = jnp.sum(_log_sigmoid(s2)) + jnp.sum(_log_sigmoid(-ns2))
    out_ref[0, 0] += part


def _tc_score(Vp, gu3, gp, gn3):
    return pl.pallas_call(
        _tc_body,
        grid=(GRID,),
        in_specs=[
            pl.BlockSpec((VDIM, D), lambda i: (0, 0)),
            pl.BlockSpec((C, BB, D), lambda i: (0, i, 0)),
            pl.BlockSpec((BB, D), lambda i: (i, 0)),
            pl.BlockSpec((K, BB, D), lambda i: (0, i, 0)),
        ],
        out_specs=pl.BlockSpec((1, 1), lambda i: (0, 0),
                               memory_space=pltpu.SMEM),
        out_shape=jax.ShapeDtypeStruct((1, 1), jnp.float32),
    )(Vp, gu3, gp, gn3)


def kernel(pos_u, pos_w, neg_w, U, W, Vp):
    iu_t = pos_u.astype(jnp.int32).T.reshape(-1)
    in_t = neg_w.astype(jnp.int32).T.reshape(-1)
    iw_t = jnp.concatenate([pos_w.astype(jnp.int32), in_t])
    gu = _sc_gather_table(C * B, U, iu_t)
    gw = _sc_gather_table(B + K * B, W, iw_t)
    out = _tc_score(Vp, gu.reshape(C, B, D), gw[:B],
                    gw[B:].reshape(K, B, D))
    return -out[0, 0]


# speedup vs baseline: 1.1655x; 1.1489x over previous
import functools

import jax
import jax.numpy as jnp
from jax import lax
from jax.experimental import pallas as pl
from jax.experimental.pallas import tpu as pltpu
from jax.experimental.pallas import tpu_sc as plsc

B, C, K = 4096, 20, 5
D, VDIM = 64, 16
G, SL = 125000, 8

NC, NS = 2, 16
NW = NC * NS
CH = 16
L = 16

NB = 8
PD = 4
NBS = 4


def _gather_rows_body(n_rows, tab_hbm, gi_hbm, out_hbm,
                      giv, buf, gsems, osems):
    per_w = n_rows // NW
    n_ch = per_w // CH
    wid = lax.axis_index("s") * NC + lax.axis_index("c")
    base = wid * per_w
    pltpu.sync_copy(gi_hbm.at[pl.ds(base, per_w)], giv)

    def fire_gather(j, b):
        gvec = giv[pl.ds(pl.multiple_of(j * CH, L), L)]
        for i in range(CH):
            pltpu.async_copy(tab_hbm.at[gvec[i]], buf.at[b, i], gsems.at[b])

    def wait_gather(b):
        pltpu.make_async_copy(tab_hbm.at[pl.ds(0, CH)], buf.at[b],
                              gsems.at[b]).wait()

    def fire_wb(j, b):
        dst = out_hbm.at[pl.ds(pl.multiple_of(base + j * CH, CH), CH)]
        return pltpu.async_copy(buf.at[b], dst, osems.at[b])

    def wait_wb(b):
        pltpu.make_async_copy(out_hbm.at[pl.ds(0, CH)], buf.at[b],
                              osems.at[b]).wait()

    for j in range(PD):
        fire_gather(j, j)

    def loop_body(j2, carry):
        for b in range(NB):
            j = j2 * NB + b
            bn = (b + PD) % NB
            wait_gather(b)
            fire_wb(j, b)

            @pl.when(j + PD < n_ch)
            def _():
                @pl.when(j + PD >= NB)
                def _():
                    wait_wb(bn)
                fire_gather(j + PD, bn)
        return carry

    lax.fori_loop(0, n_ch // NB, loop_body, 0)
    for b in range(NB):
        wait_wb(b)


def _sc_gather_table(n_rows, tab, gi):
    body = functools.partial(_gather_rows_body, n_rows)
    return pl.kernel(
        body,
        out_type=jax.ShapeDtypeStruct((n_rows, D), jnp.float32),
        mesh=plsc.VectorSubcoreMesh(core_axis_name="c", subcore_axis_name="s"),
        compiler_params=pltpu.CompilerParams(use_tc_tiling_on_sc=True,
                                             needs_layout_passes=False),
        scratch_types=[
            pltpu.VMEM((n_rows // NW,), jnp.int32),
            pltpu.VMEM((NB, CH, D), jnp.float32),
            pltpu.SemaphoreType.DMA((NB,)),
            pltpu.SemaphoreType.DMA((NB,)),
        ],
    )(tab, gi)


def _gather_slab_body(n_rows, tab_hbm, gi_hbm, si_hbm, out_hbm,
                      giv, siv, slab, obuf, gsems, osems):
    per_w = n_rows // NW
    n_ch = per_w // CH
    wid = lax.axis_index("s") * NC + lax.axis_index("c")
    base = wid * per_w
    pltpu.sync_copy(gi_hbm.at[pl.ds(base, per_w)], giv)
    pltpu.sync_copy(si_hbm.at[pl.ds(base, per_w)], siv)

    def fire_gather(j, b):
        gvec = giv[pl.ds(pl.multiple_of(j * CH, L), L)]
        for i in range(CH):
            pltpu.async_copy(tab_hbm.at[gvec[i]], slab.at[b, i], gsems.at[b])

    def wait_gather(b):
        pltpu.make_async_copy(tab_hbm.at[pl.ds(0, CH)], slab.at[b],
                              gsems.at[b]).wait()

    def wait_wb(b):
        pltpu.make_async_copy(out_hbm.at[pl.ds(0, CH)], obuf.at[b],
                              osems.at[b]).wait()

    def select(j, b):
        subs = siv[pl.ds(pl.multiple_of(j * CH, L), L)]
        for i in range(CH):
            row = slab.at[b, i, subs[i]]
            for c0 in range(0, D, L):
                obuf[b, i, pl.ds(c0, L)] = row[pl.ds(c0, L)]

    def fire_wb(j, b):
        dst = out_hbm.at[pl.ds(pl.multiple_of(base + j * CH, CH), CH)]
        return pltpu.async_copy(obuf.at[b], dst, osems.at[b])

    for j in range(NBS - 1):
        fire_gather(j, j)

    def loop_body(j2, carry):
        for b in range(NBS):
            j = j2 * NBS + b
            wait_gather(b)

            @pl.when(j + NBS - 1 < n_ch)
            def _():
                fire_gather(j + NBS - 1, (j + NBS - 1) % NBS)

            @pl.when(j2 > 0)
            def _():
                wait_wb(b)

            select(j, b)
            fire_wb(j, b)
        return carry

    lax.fori_loop(0, n_ch // NBS, loop_body, 0)
    for b in range(NBS):
        wait_wb(b)


def _sc_gather_table_slab(n_rows, tab3, gi, si):
    body = functools.partial(_gather_slab_body, n_rows)
    return pl.kernel(
        body,
        out_type=jax.ShapeDtypeStruct((n_rows, D), jnp.float32),
        mesh=plsc.VectorSubcoreMesh(core_axis_name="c", subcore_axis_name="s"),
        compiler_params=pltpu.CompilerParams(use_tc_tiling_on_sc=True,
                                             needs_layout_passes=False),
        scratch_types=[
            pltpu.VMEM((n_rows // NW,), jnp.int32),
            pltpu.VMEM((n_rows // NW,), jnp.int32),
            pltpu.VMEM((NBS, CH, SL, D), jnp.float32),
            pltpu.VMEM((NBS, CH, D), jnp.float32),
            pltpu.SemaphoreType.DMA((NBS,)),
            pltpu.SemaphoreType.DMA((NBS,)),
        ],
    )(tab3, gi, si)


BB = 256
GRID = B // BB


def _log_sigmoid(x):
    return jnp.minimum(x, 0.0) - jnp.log(1.0 + jnp.exp(-jnp.abs(x)))


def _tc_body(vp_ref, gu_ref, gp_ref, gn_ref, out_ref):
    @pl.when(pl.program_id(0) == 0)
    def _init():
        out_ref[0, 0] = 0.0

    vp = vp_ref[...]
    vp2 = vp * vp
    dn = (((1,), (1,)), ((), ()))
    S = jnp.zeros((BB, D), jnp.float32)
    acc = jnp.zeros((BB, 1), jnp.float32)
    for c in range(C):
        ec = gu_ref[c]
        t = lax.dot_general(ec, vp, dn, preferred_element_type=jnp.float32)
        t2 = lax.dot_general(ec * ec, vp2, dn, preferred_element_type=jnp.float32)
        acc = acc + jnp.sum(t * t - t2, axis=1, keepdims=True)
        S = S + ec
    fm = 0.5 * acc
    pu = S + C * fm
    s2 = jnp.sum(pu * gp_ref[...], axis=1, keepdims=True)
    nsum = jnp.zeros((BB, D), jnp.float32)
    for k in range(K):
        nsum = nsum + gn_ref[k]
    ns2 = jnp.sum(nsum * pu, axis=1, keepdims=True)
    part = jnp.sum(_log_sigmoid(s2)) + jnp.sum(_log_sigmoid(-ns2))
    out_ref[0, 0] += part


def _tc_score(Vp, gu3, gp, gn3):
    return pl.pallas_call(
        _tc_body,
        grid=(GRID,),
        in_specs=[
            pl.BlockSpec((VDIM, D), lambda i: (0, 0)),
            pl.BlockSpec((C, BB, D), lambda i: (0, i, 0)),
            pl.BlockSpec((BB, D), lambda i: (i, 0)),
            pl.BlockSpec((K, BB, D), lambda i: (0, i, 0)),
        ],
        out_specs=pl.BlockSpec((1, 1), lambda i: (0, 0),
                               memory_space=pltpu.SMEM),
        out_shape=jax.ShapeDtypeStruct((1, 1), jnp.float32),
    )(Vp, gu3, gp, gn3)


def kernel(pos_u, pos_w, neg_w, U, W, Vp):
    iu_t = pos_u.astype(jnp.int32).T.reshape(-1)
    in_t = neg_w.astype(jnp.int32).T.reshape(-1)
    iw_t = jnp.concatenate([pos_w.astype(jnp.int32), in_t])
    U3 = U.reshape(G, SL, D)
    three = jnp.int32(3)
    seven = jnp.int32(7)
    gu = _sc_gather_table_slab(C * B, U3,
                               lax.shift_right_logical(iu_t, three),
                               jnp.bitwise_and(iu_t, seven))
    gw = _sc_gather_table(B + K * B, W, iw_t)
    out = _tc_score(Vp, gu.reshape(C, B, D), gw[:B],
                    gw[B:].reshape(K, B, D))
    return -out[0, 0]


# speedup vs baseline: 1.3842x; 1.1876x over previous
import functools

import jax
import jax.numpy as jnp
from jax import lax
from jax.experimental import pallas as pl
from jax.experimental.pallas import tpu as pltpu
from jax.experimental.pallas import tpu_sc as plsc

B, C, K = 4096, 20, 5
D, VDIM = 64, 16
G, SL = 500000, 2

NC, NS = 2, 16
NW = NC * NS
CH = 16
L = 16


NB = 4


def _gather_rows_body(n_rows, tab_hbm, gi_hbm, si_hbm, out_hbm,
                      giv, siv, slab, obuf, gsems, osems):
    per_w = n_rows // NW
    n_ch = per_w // CH
    wid = lax.axis_index("s") * NC + lax.axis_index("c")
    base = wid * per_w
    pltpu.sync_copy(gi_hbm.at[pl.ds(base, per_w)], giv)
    pltpu.sync_copy(si_hbm.at[pl.ds(base, per_w)], siv)

    def fire_gather(j, b):
        for i0 in range(0, CH, L):
            gvec = giv[pl.ds(pl.multiple_of(j * CH + i0, L), L)]
            for i in range(L):
                pltpu.async_copy(tab_hbm.at[gvec[i]], slab.at[b, i0 + i],
                                 gsems.at[b])

    def wait_gather(b):
        pltpu.make_async_copy(tab_hbm.at[pl.ds(0, CH)], slab.at[b],
                              gsems.at[b]).wait()

    def wait_wb(b):
        pltpu.make_async_copy(out_hbm.at[pl.ds(0, CH)], obuf.at[b],
                              osems.at[b]).wait()

    def select(j, b):
        for i0 in range(0, CH, L):
            subs = siv[pl.ds(pl.multiple_of(j * CH + i0, L), L)]
            for i in range(L):
                row = slab.at[b, i0 + i, subs[i]]
                for c0 in range(0, D, L):
                    obuf[b, i0 + i, pl.ds(c0, L)] = row[pl.ds(c0, L)]

    def fire_wb(j, b):
        dst = out_hbm.at[pl.ds(pl.multiple_of(base + j * CH, CH), CH)]
        return pltpu.async_copy(obuf.at[b], dst, osems.at[b])

    for j in range(NB - 1):
        fire_gather(j, j)

    def loop_body(j2, carry):
        for b in range(NB):
            j = j2 * NB + b
            wait_gather(b)

            @pl.when(j + NB - 1 < n_ch)
            def _():
                fire_gather(j + NB - 1, (j + NB - 1) % NB)

            @pl.when(j2 > 0)
            def _():
                wait_wb(b)

            select(j, b)
            fire_wb(j, b)
        return carry

    lax.fori_loop(0, n_ch // NB, loop_body, 0)
    for b in range(NB):
        wait_wb(b)


def _sc_gather_table(n_rows, tab3, gi, si):
    body = functools.partial(_gather_rows_body, n_rows)
    return pl.kernel(
        body,
        out_type=jax.ShapeDtypeStruct((n_rows, D), jnp.float32),
        mesh=plsc.VectorSubcoreMesh(core_axis_name="c", subcore_axis_name="s"),
        compiler_params=pltpu.CompilerParams(use_tc_tiling_on_sc=True,
                                             needs_layout_passes=False),
        scratch_types=[
            pltpu.VMEM((n_rows // NW,), jnp.int32),
            pltpu.VMEM((n_rows // NW,), jnp.int32),
            pltpu.VMEM((NB, CH, SL, D), jnp.float32),
            pltpu.VMEM((NB, CH, D), jnp.float32),
            pltpu.SemaphoreType.DMA((NB,)),
            pltpu.SemaphoreType.DMA((NB,)),
        ],
    )(tab3, gi, si)


BB = 256
GRID = B // BB


def _log_sigmoid(x):
    return jnp.minimum(x, 0.0) - jnp.log(1.0 + jnp.exp(-jnp.abs(x)))


def _tc_body(vp_ref, gu_ref, gp_ref, gn_ref, out_ref):
    @pl.when(pl.program_id(0) == 0)
    def _init():
        out_ref[0, 0] = 0.0

    vp = vp_ref[...]
    vp2 = vp * vp
    dn = (((1,), (1,)), ((), ()))
    S = jnp.zeros((BB, D), jnp.float32)
    acc = jnp.zeros((BB, 1), jnp.float32)
    for c in range(C):
        ec = gu_ref[c]
        t = lax.dot_general(ec, vp, dn, preferred_element_type=jnp.float32)
        t2 = lax.dot_general(ec * ec, vp2, dn, preferred_element_type=jnp.float32)
        acc = acc + jnp.sum(t * t - t2, axis=1, keepdims=True)
        S = S + ec
    fm = 0.5 * acc
    pu = S + C * fm
    s2 = jnp.sum(pu * gp_ref[...], axis=1, keepdims=True)
    nsum = jnp.zeros((BB, D), jnp.float32)
    for k in range(K):
        nsum = nsum + gn_ref[k]
    ns2 = jnp.sum(nsum * pu, axis=1, keepdims=True)
    part = jnp.sum(_log_sigmoid(s2)) + jnp.sum(_log_sigmoid(-ns2))
    out_ref[0, 0] += part


def _tc_score(Vp, gu3, gp, gn3):
    return pl.pallas_call(
        _tc_body,
        grid=(GRID,),
        in_specs=[
            pl.BlockSpec((VDIM, D), lambda i: (0, 0)),
            pl.BlockSpec((C, BB, D), lambda i: (0, i, 0)),
            pl.BlockSpec((BB, D), lambda i: (i, 0)),
            pl.BlockSpec((K, BB, D), lambda i: (0, i, 0)),
        ],
        out_specs=pl.BlockSpec((1, 1), lambda i: (0, 0),
                               memory_space=pltpu.SMEM),
        out_shape=jax.ShapeDtypeStruct((1, 1), jnp.float32),
    )(Vp, gu3, gp, gn3)


def kernel(pos_u, pos_w, neg_w, U, W, Vp):
    iu_t = pos_u.astype(jnp.int32).T.reshape(-1)
    in_t = neg_w.astype(jnp.int32).T.reshape(-1)
    iw_t = jnp.concatenate([pos_w.astype(jnp.int32), in_t])
    one = jnp.int32(1)
    U3 = U.reshape(G, SL, D)
    W3 = W.reshape(G, SL, D)
    gu = _sc_gather_table(C * B, U3, lax.shift_right_logical(iu_t, one),
                          jnp.bitwise_and(iu_t, one))
    gw = _sc_gather_table(B + K * B, W3, lax.shift_right_logical(iw_t, one),
                          jnp.bitwise_and(iw_t, one))
    out = _tc_score(Vp, gu.reshape(C, B, D), gw[:B],
                    gw[B:].reshape(K, B, D))
    return -out[0, 0]
